# baseline (speedup 1.0000x reference)
import functools

import jax
import jax.numpy as jnp
from jax import lax
from jax.experimental import pallas as pl
from jax.experimental.pallas import tpu as pltpu
from jax.experimental.pallas import tpu_sc as plsc

N_USERS = 25000
N_ITEMS = 25000
N_NODES = N_USERS + N_ITEMS
D = 64
E = 800000
HALF = 25000
TRASH = HALF
ACC_ROWS = 25008

NC = 2
NS = 16
LANES = 16

EROW = 128
ROWS_PER_TILE = 396
E_PAD = NS * ROWS_PER_TILE * EROW
RING = 3
G_PER_CH = 12
N_CH = ROWS_PER_TILE // G_PER_CH
WB = 1568
QLEN = 272

_mesh = plsc.VectorSubcoreMesh(core_axis_name="c", subcore_axis_name="s")


@functools.partial(
    pl.kernel,
    out_type=(
        jax.ShapeDtypeStruct((NC, NS * ROWS_PER_TILE, 2, EROW), jnp.int32),
        jax.ShapeDtypeStruct((NC, NS * ROWS_PER_TILE, EROW), jnp.float32),
        jax.ShapeDtypeStruct((NC, NS, LANES), jnp.int32),
    ),
    mesh=_mesh,
    scratch_types=[
        pltpu.VMEM((G_PER_CH, 2, EROW), jnp.int32),
        pltpu.VMEM((G_PER_CH, EROW), jnp.float32),
        pltpu.VMEM((QLEN,), jnp.int32),
        pltpu.VMEM((QLEN,), jnp.int32),
        pltpu.VMEM((QLEN,), jnp.float32),
    ],
    compiler_params=pltpu.CompilerParams(use_tc_tiling_on_sc=False,
                                         needs_layout_passes=False),
)
def _partition(eidx, evals, peidx, pvals, pcount, ibuf, vbuf, qcol, qlr, qval):
    c = lax.axis_index("c")
    s = lax.axis_index("s")
    half_base = c * HALF
    obase = s * ROWS_PER_TILE

    def _flush(qn, orow):
        pltpu.sync_copy(qcol.at[pl.ds(0, EROW)], peidx.at[c, obase + orow, 0])
        pltpu.sync_copy(qlr.at[pl.ds(0, EROW)], peidx.at[c, obase + orow, 1])
        pltpu.sync_copy(qval.at[pl.ds(0, EROW)], pvals.at[c, obase + orow])
        for t in range(EROW // LANES):
            src = pl.ds(EROW + t * LANES, LANES)
            dst = pl.ds(t * LANES, LANES)
            qcol[dst] = qcol[src]
            qlr[dst] = qlr[src]
            qval[dst] = qval[src]
        return qn - EROW, orow + 1

    def _chunk(ci, carry):
        qn, orow = carry
        cbase = s * ROWS_PER_TILE + ci * G_PER_CH
        pltpu.sync_copy(eidx.at[pl.ds(cbase, G_PER_CH)], ibuf)
        pltpu.sync_copy(evals.at[pl.ds(cbase, G_PER_CH)], vbuf)
        for jr in range(G_PER_CH):
            for t in range(EROW // LANES):
                seg = pl.ds(t * LANES, LANES)
                colv = ibuf[jr, 0, seg]
                lr = ibuf[jr, 1, seg] - half_base
                valv = vbuf[jr, seg]
                ok = (lr >= 0) & (lr < HALF)
                key = (colv << 15) | (lr & 0x7FFF)
                sk, sv, _ = plsc.sort_key_val(key, valv, mask=ok)
                qcol[pl.ds(qn, LANES)] = sk >> 15
                qlr[pl.ds(qn, LANES)] = sk & 0x7FFF
                qval[pl.ds(qn, LANES)] = sv
                qn = qn + plsc.all_reduce_population_count(ok)[0]
            qn, orow = lax.cond(qn >= EROW, _flush,
                                lambda a, b: (a, b), qn, orow)
        return qn, orow

    qn, orow = lax.fori_loop(0, N_CH, _chunk, (0, 0))

    lanes = jnp.arange(LANES, dtype=jnp.int32)
    for t in range(EROW // LANES):
        seg = pl.ds(t * LANES, LANES)
        keep = (lanes + t * LANES) < qn
        qcol[seg] = jnp.where(keep, qcol[seg], 0)
        qlr[seg] = jnp.where(keep, qlr[seg], TRASH)
        qval[seg] = jnp.where(keep, qval[seg], 0.0)
    qn, orow = lax.cond(qn > 0, _flush, lambda a, b: (a, b), qn, orow)

    qcol[pl.ds(0, LANES)] = jnp.full((LANES,), orow, jnp.int32)
    pltpu.sync_copy(qcol.at[pl.ds(0, LANES)], pcount.at[c, s])


@functools.partial(
    pl.kernel,
    out_type=jax.ShapeDtypeStruct((N_NODES, D), jnp.float32),
    mesh=_mesh,
    scratch_types=[
        pltpu.VMEM((RING * EROW, D), jnp.float32),
        pltpu.VMEM((G_PER_CH, 2, EROW), jnp.int32),
        pltpu.VMEM((G_PER_CH, EROW), jnp.float32),
        pltpu.VMEM((LANES,), jnp.int32),
        pltpu.VMEM_SHARED((ACC_ROWS, D), jnp.float32),
        pltpu.SemaphoreType.DMA,
        pltpu.SemaphoreType.DMA,
    ],
    compiler_params=pltpu.CompilerParams(use_tc_tiling_on_sc=False),
)
def _sc_layer(emb, peidx, pvals, pcount, out, gbuf, ibuf, vbuf, nrv, acc,
              gsem, ssem):
    c = lax.axis_index("c")
    s = lax.axis_index("s")
    half_base = c * HALF

    pltpu.sync_copy(pcount.at[c, s], nrv)
    nr = nrv[...][0]

    zeros = jnp.zeros((LANES,), jnp.float32)

    def _zrow(r, carry):
        for k in range(D // LANES):
            gbuf[r, pl.ds(k * LANES, LANES)] = zeros
        return carry

    lax.fori_loop(0, RING * EROW, _zrow, 0)
    zbase = jnp.minimum(s * WB, ACC_ROWS - WB)
    nfull = WB // (RING * EROW)
    for i in range(nfull):
        pltpu.sync_copy(gbuf, acc.at[pl.ds(zbase + i * RING * EROW,
                                           RING * EROW)])
    rem = WB - nfull * RING * EROW
    if rem:
        pltpu.sync_copy(gbuf.at[pl.ds(0, rem)],
                        acc.at[pl.ds(zbase + WB - rem, rem)])
    plsc.subcore_barrier()

    def _drain(sem, ph):
        pltpu.make_async_copy(emb.at[pl.ds(0, EROW)],
                              gbuf.at[pl.ds(ph * EROW, EROW)], sem).wait()

    def _process(jr, q):
        _drain(gsem, q)

        def _scale(t, carry):
            vv = vbuf[jr, pl.ds(t * LANES, LANES)]
            for l in range(LANES):
                v = vv[l]
                r = q * EROW + t * LANES + l
                for k in range(D // LANES):
                    gbuf[r, pl.ds(k * LANES, LANES)] = (
                        gbuf[r, pl.ds(k * LANES, LANES)] * v)
            return carry

        lax.fori_loop(0, EROW // LANES, _scale, 0, unroll=4)
        pltpu.async_copy(gbuf.at[pl.ds(q * EROW, EROW)],
                         acc.at[ibuf.at[jr, 1]], ssem, add=True)

    def _chunk(ci, carry):
        @pl.when((ci >= 1) & (ci * G_PER_CH - 1 < nr))
        def _():
            _process(G_PER_CH - 1, (G_PER_CH - 1) % RING)

        @pl.when(ci * G_PER_CH < nr)
        def _():
            cbase = s * ROWS_PER_TILE + ci * G_PER_CH
            pltpu.sync_copy(peidx.at[c, pl.ds(cbase, G_PER_CH)], ibuf)
            pltpu.sync_copy(pvals.at[c, pl.ds(cbase, G_PER_CH)], vbuf)

        for pg in range(G_PER_CH):
            g = ci * G_PER_CH + pg
            ph = pg % RING

            @pl.when((g >= RING) & (g < nr))
            def _():
                _drain(ssem, ph)

            @pl.when(g < nr)
            def _():
                pltpu.async_copy(emb.at[ibuf.at[pg, 0]],
                                 gbuf.at[pl.ds(ph * EROW, EROW)], gsem)
            if pg >= 1:
                @pl.when(g - 1 < nr)
                def _():
                    _process(pg - 1, (pg - 1) % RING)
        return carry

    lax.fori_loop(0, N_CH, _chunk, 0)

    @pl.when(nr >= ROWS_PER_TILE)
    def _():
        _process(G_PER_CH - 1, (G_PER_CH - 1) % RING)

    for i in range(RING):
        @pl.when(nr >= i + 1)
        def _():
            _drain(ssem, i)
    plsc.subcore_barrier()

    wbase = jnp.minimum(s * WB, HALF - WB)
    pltpu.sync_copy(acc.at[pl.ds(wbase, WB)],
                    out.at[pl.ds(half_base + wbase, WB)])


def _mean_body(a_ref, b_ref, c_ref, d_ref, o_ref):
    o_ref[...] = 0.25 * (a_ref[...] + b_ref[...] + c_ref[...] + d_ref[...])


def _mean4(e0, e1, e2, e3):
    blk = 2000
    grid = N_NODES // blk
    spec = pl.BlockSpec((blk, D), lambda i: (i, 0))
    return pl.pallas_call(
        _mean_body,
        grid=(grid,),
        in_specs=[spec, spec, spec, spec],
        out_specs=spec,
        out_shape=jax.ShapeDtypeStruct((N_NODES, D), jnp.float32),
    )(e0, e1, e2, e3)


def _lm_body(phi3_ref, phi4_ref, phi2_ref, ulm_ref, o_ref):
    w = jnp.dot(phi4_ref[...], ulm_ref[...],
                preferred_element_type=jnp.float32)
    o_ref[...] = 0.1 * jnp.dot(phi3_ref[...], w,
                               preferred_element_type=jnp.float32) \
        + 0.9 * jnp.dot(phi2_ref[...], ulm_ref[...],
                        preferred_element_type=jnp.float32)


def _user_lm(phi3, phi4, phi2, user_lm_T):
    blk = 1000
    grid = N_USERS // blk
    k_sel = phi2.shape[1]
    r = phi3.shape[1]
    return pl.pallas_call(
        _lm_body,
        grid=(grid,),
        in_specs=[
            pl.BlockSpec((blk, r), lambda i: (i, 0)),
            pl.BlockSpec((r, k_sel), lambda i: (0, 0)),
            pl.BlockSpec((blk, k_sel), lambda i: (i, 0)),
            pl.BlockSpec((k_sel, D), lambda i: (0, 0)),
        ],
        out_specs=pl.BlockSpec((blk, D), lambda i: (i, 0)),
        out_shape=jax.ShapeDtypeStruct((N_USERS, D), jnp.float32),
    )(phi3, phi4, phi2, user_lm_T)


def kernel(user_emb, item_emb, values, user_lm_T, item_lm, phi2, phi3, phi4,
           edge_index):
    rows = edge_index[0].astype(jnp.int32)
    cols = edge_index[1].astype(jnp.int32)
    pad = E_PAD - E
    rows_p = jnp.concatenate(
        [rows, jnp.full((pad,), -1, jnp.int32)]).reshape(-1, EROW)
    cols_p = jnp.concatenate(
        [cols, jnp.zeros((pad,), jnp.int32)]).reshape(-1, EROW)
    vals_p = jnp.concatenate(
        [values.astype(jnp.float32),
         jnp.zeros((pad,), jnp.float32)]).reshape(-1, EROW)
    eidx = jnp.stack([cols_p, rows_p], axis=1)

    peidx, pvals, pcount = _partition(eidx, vals_p)

    e0 = jnp.concatenate([user_emb, item_emb], axis=0)
    e1 = _sc_layer(e0, peidx, pvals, pcount)
    e2 = _sc_layer(e1, peidx, pvals, pcount)
    e3 = _sc_layer(e2, peidx, pvals, pcount)

    light = _mean4(e0, e1, e2, e3)
    users_id_emb = light[:N_USERS]
    items_id_emb = light[N_USERS:]
    user_lm = _user_lm(phi3, phi4, phi2, user_lm_T)
    return (users_id_emb, items_id_emb, user_lm, item_lm)


# speedup vs baseline: 1.0497x; 1.0497x over previous
import functools

import jax
import jax.numpy as jnp
from jax import lax
from jax.experimental import pallas as pl
from jax.experimental.pallas import tpu as pltpu
from jax.experimental.pallas import tpu_sc as plsc

N_USERS = 25000
N_ITEMS = 25000
N_NODES = N_USERS + N_ITEMS
D = 64
E = 800000
HALF = 25000
TRASH = HALF
ACC_ROWS = 25008

NC = 2
NS = 16
LANES = 16

EROW = 128
ROWS_PER_TILE = 396
E_PAD = NS * ROWS_PER_TILE * EROW
RING = 3
G_PER_CH = 12
N_CH = ROWS_PER_TILE // G_PER_CH
WB = 1568
QLEN = 272

_mesh = plsc.VectorSubcoreMesh(core_axis_name="c", subcore_axis_name="s")


@functools.partial(
    pl.kernel,
    out_type=(
        jax.ShapeDtypeStruct((NC, NS * ROWS_PER_TILE, 2, EROW), jnp.int32),
        jax.ShapeDtypeStruct((NC, NS * ROWS_PER_TILE, EROW), jnp.float32),
        jax.ShapeDtypeStruct((NC, NS, LANES), jnp.int32),
    ),
    mesh=_mesh,
    scratch_types=[
        pltpu.VMEM((G_PER_CH, 2, EROW), jnp.int32),
        pltpu.VMEM((G_PER_CH, EROW), jnp.float32),
        pltpu.VMEM((QLEN,), jnp.int32),
        pltpu.VMEM((QLEN,), jnp.int32),
        pltpu.VMEM((QLEN,), jnp.float32),
    ],
    compiler_params=pltpu.CompilerParams(use_tc_tiling_on_sc=False,
                                         needs_layout_passes=False),
)
def _partition(eidx, evals, peidx, pvals, pcount, ibuf, vbuf, qcol, qlr, qval):
    c = lax.axis_index("c")
    s = lax.axis_index("s")
    half_base = c * HALF
    obase = s * ROWS_PER_TILE

    def _flush(qn, orow):
        pltpu.sync_copy(qcol.at[pl.ds(0, EROW)], peidx.at[c, obase + orow, 0])
        pltpu.sync_copy(qlr.at[pl.ds(0, EROW)], peidx.at[c, obase + orow, 1])
        pltpu.sync_copy(qval.at[pl.ds(0, EROW)], pvals.at[c, obase + orow])
        for t in range(EROW // LANES):
            src = pl.ds(EROW + t * LANES, LANES)
            dst = pl.ds(t * LANES, LANES)
            qcol[dst] = qcol[src]
            qlr[dst] = qlr[src]
            qval[dst] = qval[src]
        return qn - EROW, orow + 1

    def _chunk(ci, carry):
        qn, orow = carry
        cbase = s * ROWS_PER_TILE + ci * G_PER_CH
        pltpu.sync_copy(eidx.at[pl.ds(cbase, G_PER_CH)], ibuf)
        pltpu.sync_copy(evals.at[pl.ds(cbase, G_PER_CH)], vbuf)
        for jr in range(G_PER_CH):
            for t in range(EROW // LANES):
                seg = pl.ds(t * LANES, LANES)
                colv = ibuf[jr, 0, seg]
                lr = ibuf[jr, 1, seg] - half_base
                valv = vbuf[jr, seg]
                ok = (lr >= 0) & (lr < HALF)
                key = (colv << 15) | (lr & 0x7FFF)
                sk, sv, _ = plsc.sort_key_val(key, valv, mask=ok)
                qcol[pl.ds(qn, LANES)] = sk >> 15
                qlr[pl.ds(qn, LANES)] = sk & 0x7FFF
                qval[pl.ds(qn, LANES)] = sv
                qn = qn + plsc.all_reduce_population_count(ok)[0]
            qn, orow = lax.cond(qn >= EROW, _flush,
                                lambda a, b: (a, b), qn, orow)
        return qn, orow

    qn, orow = lax.fori_loop(0, N_CH, _chunk, (0, 0))

    lanes = jnp.arange(LANES, dtype=jnp.int32)
    for t in range(EROW // LANES):
        seg = pl.ds(t * LANES, LANES)
        keep = (lanes + t * LANES) < qn
        qcol[seg] = jnp.where(keep, qcol[seg], 0)
        qlr[seg] = jnp.where(keep, qlr[seg], TRASH)
        qval[seg] = jnp.where(keep, qval[seg], 0.0)
    qn, orow = lax.cond(qn > 0, _flush, lambda a, b: (a, b), qn, orow)

    qcol[pl.ds(0, LANES)] = jnp.full((LANES,), orow, jnp.int32)
    pltpu.sync_copy(qcol.at[pl.ds(0, LANES)], pcount.at[c, s])


@functools.partial(
    pl.kernel,
    out_type=jax.ShapeDtypeStruct((N_NODES, D), jnp.float32),
    mesh=_mesh,
    scratch_types=[
        pltpu.VMEM((RING * EROW, D), jnp.float32),
        pltpu.VMEM((G_PER_CH, 2, EROW), jnp.int32),
        pltpu.VMEM((G_PER_CH, EROW), jnp.float32),
        pltpu.VMEM((LANES,), jnp.int32),
        pltpu.VMEM_SHARED((ACC_ROWS, D), jnp.float32),
        pltpu.SemaphoreType.DMA,
        pltpu.SemaphoreType.DMA,
    ],
    compiler_params=pltpu.CompilerParams(use_tc_tiling_on_sc=False),
)
def _sc_layer(emb, peidx, pvals, pcount, out, gbuf, ibuf, vbuf, nrv, acc,
              gsem, ssem):
    c = lax.axis_index("c")
    s = lax.axis_index("s")
    half_base = c * HALF

    pltpu.sync_copy(pcount.at[c, s], nrv)
    nr = nrv[...][0]

    zeros = jnp.zeros((LANES,), jnp.float32)

    def _zrow(r, carry):
        for k in range(D // LANES):
            gbuf[r, pl.ds(k * LANES, LANES)] = zeros
        return carry

    lax.fori_loop(0, RING * EROW, _zrow, 0)
    zbase = jnp.minimum(s * WB, ACC_ROWS - WB)
    nfull = WB // (RING * EROW)
    for i in range(nfull):
        pltpu.sync_copy(gbuf, acc.at[pl.ds(zbase + i * RING * EROW,
                                           RING * EROW)])
    rem = WB - nfull * RING * EROW
    if rem:
        pltpu.sync_copy(gbuf.at[pl.ds(0, rem)],
                        acc.at[pl.ds(zbase + WB - rem, rem)])
    plsc.subcore_barrier()

    def _drain(sem, ph):
        pltpu.make_async_copy(emb.at[pl.ds(0, EROW)],
                              gbuf.at[pl.ds(ph * EROW, EROW)], sem).wait()

    def _process(jr, q):
        _drain(gsem, q)

        def _scale(t, carry):
            vv = vbuf[jr, pl.ds(t * LANES, LANES)]
            for l in range(LANES):
                v = vv[l]
                r = q * EROW + t * LANES + l
                for k in range(D // LANES):
                    gbuf[r, pl.ds(k * LANES, LANES)] = (
                        gbuf[r, pl.ds(k * LANES, LANES)] * v)
            return carry

        lax.fori_loop(0, EROW // LANES, _scale, 0, unroll=2)
        pltpu.async_copy(gbuf.at[pl.ds(q * EROW, EROW)],
                         acc.at[ibuf.at[jr, 1]], ssem, add=True)

    def _chunk(ci, carry):
        @pl.when((ci >= 1) & (ci * G_PER_CH - 1 < nr))
        def _():
            _process(G_PER_CH - 1, (G_PER_CH - 1) % RING)

        @pl.when(ci * G_PER_CH < nr)
        def _():
            cbase = s * ROWS_PER_TILE + ci * G_PER_CH
            pltpu.sync_copy(peidx.at[c, pl.ds(cbase, G_PER_CH)], ibuf)
            pltpu.sync_copy(pvals.at[c, pl.ds(cbase, G_PER_CH)], vbuf)

        for pg in range(G_PER_CH):
            g = ci * G_PER_CH + pg
            ph = pg % RING

            @pl.when((g >= RING) & (g < nr))
            def _():
                _drain(ssem, ph)

            @pl.when(g < nr)
            def _():
                pltpu.async_copy(emb.at[ibuf.at[pg, 0]],
                                 gbuf.at[pl.ds(ph * EROW, EROW)], gsem)
            if pg >= 1:
                @pl.when(g - 1 < nr)
                def _():
                    _process(pg - 1, (pg - 1) % RING)
        return carry

    lax.fori_loop(0, N_CH, _chunk, 0)

    @pl.when(nr >= ROWS_PER_TILE)
    def _():
        _process(G_PER_CH - 1, (G_PER_CH - 1) % RING)

    for i in range(RING):
        @pl.when(nr >= i + 1)
        def _():
            _drain(ssem, i)
    plsc.subcore_barrier()

    wbase = jnp.minimum(s * WB, HALF - WB)
    pltpu.sync_copy(acc.at[pl.ds(wbase, WB)],
                    out.at[pl.ds(half_base + wbase, WB)])


def _mean_body(a_ref, b_ref, c_ref, d_ref, o_ref):
    o_ref[...] = 0.25 * (a_ref[...] + b_ref[...] + c_ref[...] + d_ref[...])


def _mean4(e0, e1, e2, e3):
    blk = 2000
    grid = N_NODES // blk
    spec = pl.BlockSpec((blk, D), lambda i: (i, 0))
    return pl.pallas_call(
        _mean_body,
        grid=(grid,),
        in_specs=[spec, spec, spec, spec],
        out_specs=spec,
        out_shape=jax.ShapeDtypeStruct((N_NODES, D), jnp.float32),
    )(e0, e1, e2, e3)


def _lm_body(phi3_ref, phi4_ref, phi2_ref, ulm_ref, o_ref):
    w = jnp.dot(phi4_ref[...], ulm_ref[...],
                preferred_element_type=jnp.float32)
    o_ref[...] = 0.1 * jnp.dot(phi3_ref[...], w,
                               preferred_element_type=jnp.float32) \
        + 0.9 * jnp.dot(phi2_ref[...], ulm_ref[...],
                        preferred_element_type=jnp.float32)


def _user_lm(phi3, phi4, phi2, user_lm_T):
    blk = 1000
    grid = N_USERS // blk
    k_sel = phi2.shape[1]
    r = phi3.shape[1]
    return pl.pallas_call(
        _lm_body,
        grid=(grid,),
        in_specs=[
            pl.BlockSpec((blk, r), lambda i: (i, 0)),
            pl.BlockSpec((r, k_sel), lambda i: (0, 0)),
            pl.BlockSpec((blk, k_sel), lambda i: (i, 0)),
            pl.BlockSpec((k_sel, D), lambda i: (0, 0)),
        ],
        out_specs=pl.BlockSpec((blk, D), lambda i: (i, 0)),
        out_shape=jax.ShapeDtypeStruct((N_USERS, D), jnp.float32),
    )(phi3, phi4, phi2, user_lm_T)


def kernel(user_emb, item_emb, values, user_lm_T, item_lm, phi2, phi3, phi4,
           edge_index):
    rows = edge_index[0].astype(jnp.int32)
    cols = edge_index[1].astype(jnp.int32)
    pad = E_PAD - E
    rows_p = jnp.concatenate(
        [rows, jnp.full((pad,), -1, jnp.int32)]).reshape(-1, EROW)
    cols_p = jnp.concatenate(
        [cols, jnp.zeros((pad,), jnp.int32)]).reshape(-1, EROW)
    vals_p = jnp.concatenate(
        [values.astype(jnp.float32),
         jnp.zeros((pad,), jnp.float32)]).reshape(-1, EROW)
    eidx = jnp.stack([cols_p, rows_p], axis=1)

    peidx, pvals, pcount = _partition(eidx, vals_p)

    e0 = jnp.concatenate([user_emb, item_emb], axis=0)
    e1 = _sc_layer(e0, peidx, pvals, pcount)
    e2 = _sc_layer(e1, peidx, pvals, pcount)
    e3 = _sc_layer(e2, peidx, pvals, pcount)

    light = _mean4(e0, e1, e2, e3)
    users_id_emb = light[:N_USERS]
    items_id_emb = light[N_USERS:]
    user_lm = _user_lm(phi3, phi4, phi2, user_lm_T)
    return (users_id_emb, items_id_emb, user_lm, item_lm)


# speedup vs baseline: 1.0555x; 1.0056x over previous
import functools

import jax
import jax.numpy as jnp
from jax import lax
from jax.experimental import pallas as pl
from jax.experimental.pallas import tpu as pltpu
from jax.experimental.pallas import tpu_sc as plsc

N_USERS = 25000
N_ITEMS = 25000
N_NODES = N_USERS + N_ITEMS
D = 64
E = 800000
HALF = 25000
TRASH = HALF
ACC_ROWS = 25008

NC = 2
NS = 16
LANES = 16

EROW = 128
ROWS_PER_TILE = 396
E_PAD = NS * ROWS_PER_TILE * EROW
RING = 3
G_PER_CH = 12
N_CH = ROWS_PER_TILE // G_PER_CH
WB = 1568
QLEN = 528

_mesh = plsc.VectorSubcoreMesh(core_axis_name="c", subcore_axis_name="s")


@functools.partial(
    pl.kernel,
    out_type=(
        jax.ShapeDtypeStruct((NC, NS * ROWS_PER_TILE * EROW), jnp.int32),
        jax.ShapeDtypeStruct((NC, NS * ROWS_PER_TILE * EROW), jnp.int32),
        jax.ShapeDtypeStruct((NC, NS * ROWS_PER_TILE * EROW), jnp.float32),
        jax.ShapeDtypeStruct((NC, NS, LANES), jnp.int32),
    ),
    mesh=_mesh,
    scratch_types=[
        pltpu.VMEM((G_PER_CH, 2, EROW), jnp.int32),
        pltpu.VMEM((G_PER_CH, EROW), jnp.float32),
        pltpu.VMEM((QLEN,), jnp.int32),
        pltpu.VMEM((QLEN,), jnp.int32),
        pltpu.VMEM((QLEN,), jnp.float32),
    ],
    compiler_params=pltpu.CompilerParams(use_tc_tiling_on_sc=False,
                                         needs_layout_passes=False),
)
def _partition(eidx, evals, pcol, plr, pval, pcount,
               ibuf, vbuf, qcol, qlr, qval):
    c = lax.axis_index("c")
    s = lax.axis_index("s")
    half_base = c * HALF
    obase = s * ROWS_PER_TILE

    def _flush2(qn, orow):
        off = (obase + orow) * EROW
        pltpu.sync_copy(qcol.at[pl.ds(0, 2 * EROW)],
                        pcol.at[c, pl.ds(off, 2 * EROW)])
        pltpu.sync_copy(qlr.at[pl.ds(0, 2 * EROW)],
                        plr.at[c, pl.ds(off, 2 * EROW)])
        pltpu.sync_copy(qval.at[pl.ds(0, 2 * EROW)],
                        pval.at[c, pl.ds(off, 2 * EROW)])
        for t in range(2 * EROW // LANES):
            src = pl.ds(2 * EROW + t * LANES, LANES)
            dst = pl.ds(t * LANES, LANES)
            qcol[dst] = qcol[src]
            qlr[dst] = qlr[src]
            qval[dst] = qval[src]
        return qn - 2 * EROW, orow + 2

    def _chunk(ci, carry):
        qn, orow = carry
        cbase = s * ROWS_PER_TILE + ci * G_PER_CH
        pltpu.sync_copy(eidx.at[pl.ds(cbase, G_PER_CH)], ibuf)
        pltpu.sync_copy(evals.at[pl.ds(cbase, G_PER_CH)], vbuf)
        for jr in range(G_PER_CH):
            for t in range(EROW // LANES):
                seg = pl.ds(t * LANES, LANES)
                colv = ibuf[jr, 0, seg]
                lr = ibuf[jr, 1, seg] - half_base
                valv = vbuf[jr, seg]
                ok = (lr >= 0) & (lr < HALF)
                key = (colv << 15) | (lr & 0x7FFF)
                sk, sv, _ = plsc.sort_key_val(key, valv, mask=ok)
                qcol[pl.ds(qn, LANES)] = sk >> 15
                qlr[pl.ds(qn, LANES)] = sk & 0x7FFF
                qval[pl.ds(qn, LANES)] = sv
                qn = qn + plsc.all_reduce_population_count(ok)[0]
            if jr % 2 == 1:
                qn, orow = lax.cond(qn >= 2 * EROW, _flush2,
                                    lambda a, b: (a, b), qn, orow)
        return qn, orow

    qn, orow = lax.fori_loop(0, N_CH, _chunk, (0, 0))

    lanes = jnp.arange(LANES, dtype=jnp.int32)
    for t in range(2 * EROW // LANES):
        seg = pl.ds(t * LANES, LANES)
        keep = (lanes + t * LANES) < qn
        qcol[seg] = jnp.where(keep, qcol[seg], 0)
        qlr[seg] = jnp.where(keep, qlr[seg], TRASH)
        qval[seg] = jnp.where(keep, qval[seg], 0.0)

    def _flush_last(base, orow_):
        off = (obase + orow_) * EROW
        pltpu.sync_copy(qcol.at[pl.ds(base * EROW, EROW)],
                        pcol.at[c, pl.ds(off, EROW)])
        pltpu.sync_copy(qlr.at[pl.ds(base * EROW, EROW)],
                        plr.at[c, pl.ds(off, EROW)])
        pltpu.sync_copy(qval.at[pl.ds(base * EROW, EROW)],
                        pval.at[c, pl.ds(off, EROW)])
        return orow_ + 1

    orow = lax.cond(qn > 0, lambda o: _flush_last(0, o),
                    lambda o: o, orow)
    orow = lax.cond(qn > EROW, lambda o: _flush_last(1, o),
                    lambda o: o, orow)

    qcol[pl.ds(0, LANES)] = jnp.full((LANES,), orow, jnp.int32)
    pltpu.sync_copy(qcol.at[pl.ds(0, LANES)], pcount.at[c, s])


@functools.partial(
    pl.kernel,
    out_type=jax.ShapeDtypeStruct((N_NODES, D), jnp.float32),
    mesh=_mesh,
    scratch_types=[
        pltpu.VMEM((RING * EROW, D), jnp.float32),
        pltpu.VMEM((G_PER_CH, 2, EROW), jnp.int32),
        pltpu.VMEM((G_PER_CH, EROW), jnp.float32),
        pltpu.VMEM((LANES,), jnp.int32),
        pltpu.VMEM_SHARED((ACC_ROWS, D), jnp.float32),
        pltpu.SemaphoreType.DMA,
        pltpu.SemaphoreType.DMA,
    ],
    compiler_params=pltpu.CompilerParams(use_tc_tiling_on_sc=False),
)
def _sc_layer(emb, peidx, pvals, pcount, out, gbuf, ibuf, vbuf, nrv, acc,
              gsem, ssem):
    c = lax.axis_index("c")
    s = lax.axis_index("s")
    half_base = c * HALF

    pltpu.sync_copy(pcount.at[c, s], nrv)
    nr = nrv[...][0]

    zeros = jnp.zeros((LANES,), jnp.float32)

    def _zrow(r, carry):
        for k in range(D // LANES):
            gbuf[r, pl.ds(k * LANES, LANES)] = zeros
        return carry

    lax.fori_loop(0, RING * EROW, _zrow, 0)
    zbase = jnp.minimum(s * WB, ACC_ROWS - WB)
    nfull = WB // (RING * EROW)
    for i in range(nfull):
        pltpu.sync_copy(gbuf, acc.at[pl.ds(zbase + i * RING * EROW,
                                           RING * EROW)])
    rem = WB - nfull * RING * EROW
    if rem:
        pltpu.sync_copy(gbuf.at[pl.ds(0, rem)],
                        acc.at[pl.ds(zbase + WB - rem, rem)])
    plsc.subcore_barrier()

    def _drain(sem, ph):
        pltpu.make_async_copy(emb.at[pl.ds(0, EROW)],
                              gbuf.at[pl.ds(ph * EROW, EROW)], sem).wait()

    def _process(jr, q):
        _drain(gsem, q)

        def _scale(t, carry):
            vv = vbuf[jr, pl.ds(t * LANES, LANES)]
            for l in range(LANES):
                v = vv[l]
                r = q * EROW + t * LANES + l
                for k in range(D // LANES):
                    gbuf[r, pl.ds(k * LANES, LANES)] = (
                        gbuf[r, pl.ds(k * LANES, LANES)] * v)
            return carry

        lax.fori_loop(0, EROW // LANES, _scale, 0, unroll=2)
        pltpu.async_copy(gbuf.at[pl.ds(q * EROW, EROW)],
                         acc.at[ibuf.at[jr, 1]], ssem, add=True)

    def _chunk(ci, carry):
        @pl.when((ci >= 1) & (ci * G_PER_CH - 1 < nr))
        def _():
            _process(G_PER_CH - 1, (G_PER_CH - 1) % RING)

        @pl.when(ci * G_PER_CH < nr)
        def _():
            cbase = s * ROWS_PER_TILE + ci * G_PER_CH
            pltpu.sync_copy(peidx.at[c, pl.ds(cbase, G_PER_CH)], ibuf)
            pltpu.sync_copy(pvals.at[c, pl.ds(cbase, G_PER_CH)], vbuf)

        for pg in range(G_PER_CH):
            g = ci * G_PER_CH + pg
            ph = pg % RING

            @pl.when((g >= RING) & (g < nr))
            def _():
                _drain(ssem, ph)

            @pl.when(g < nr)
            def _():
                pltpu.async_copy(emb.at[ibuf.at[pg, 0]],
                                 gbuf.at[pl.ds(ph * EROW, EROW)], gsem)
            if pg >= 1:
                @pl.when(g - 1 < nr)
                def _():
                    _process(pg - 1, (pg - 1) % RING)
        return carry

    lax.fori_loop(0, N_CH, _chunk, 0)

    @pl.when(nr >= ROWS_PER_TILE)
    def _():
        _process(G_PER_CH - 1, (G_PER_CH - 1) % RING)

    for i in range(RING):
        @pl.when(nr >= i + 1)
        def _():
            _drain(ssem, i)
    plsc.subcore_barrier()

    wbase = jnp.minimum(s * WB, HALF - WB)
    pltpu.sync_copy(acc.at[pl.ds(wbase, WB)],
                    out.at[pl.ds(half_base + wbase, WB)])


def _mean_body(a_ref, b_ref, c_ref, d_ref, o_ref):
    o_ref[...] = 0.25 * (a_ref[...] + b_ref[...] + c_ref[...] + d_ref[...])


def _mean4(e0, e1, e2, e3):
    blk = 2000
    grid = N_NODES // blk
    spec = pl.BlockSpec((blk, D), lambda i: (i, 0))
    return pl.pallas_call(
        _mean_body,
        grid=(grid,),
        in_specs=[spec, spec, spec, spec],
        out_specs=spec,
        out_shape=jax.ShapeDtypeStruct((N_NODES, D), jnp.float32),
    )(e0, e1, e2, e3)


def _lm_body(phi3_ref, phi4_ref, phi2_ref, ulm_ref, o_ref):
    w = jnp.dot(phi4_ref[...], ulm_ref[...],
                preferred_element_type=jnp.float32)
    o_ref[...] = 0.1 * jnp.dot(phi3_ref[...], w,
                               preferred_element_type=jnp.float32) \
        + 0.9 * jnp.dot(phi2_ref[...], ulm_ref[...],
                        preferred_element_type=jnp.float32)


def _user_lm(phi3, phi4, phi2, user_lm_T):
    blk = 1000
    grid = N_USERS // blk
    k_sel = phi2.shape[1]
    r = phi3.shape[1]
    return pl.pallas_call(
        _lm_body,
        grid=(grid,),
        in_specs=[
            pl.BlockSpec((blk, r), lambda i: (i, 0)),
            pl.BlockSpec((r, k_sel), lambda i: (0, 0)),
            pl.BlockSpec((blk, k_sel), lambda i: (i, 0)),
            pl.BlockSpec((k_sel, D), lambda i: (0, 0)),
        ],
        out_specs=pl.BlockSpec((blk, D), lambda i: (i, 0)),
        out_shape=jax.ShapeDtypeStruct((N_USERS, D), jnp.float32),
    )(phi3, phi4, phi2, user_lm_T)


def kernel(user_emb, item_emb, values, user_lm_T, item_lm, phi2, phi3, phi4,
           edge_index):
    rows = edge_index[0].astype(jnp.int32)
    cols = edge_index[1].astype(jnp.int32)
    pad = E_PAD - E
    rows_p = jnp.concatenate(
        [rows, jnp.full((pad,), -1, jnp.int32)]).reshape(-1, EROW)
    cols_p = jnp.concatenate(
        [cols, jnp.zeros((pad,), jnp.int32)]).reshape(-1, EROW)
    vals_p = jnp.concatenate(
        [values.astype(jnp.float32),
         jnp.zeros((pad,), jnp.float32)]).reshape(-1, EROW)
    eidx = jnp.stack([cols_p, rows_p], axis=1)

    pcol, plr, pval, pcount = _partition(eidx, vals_p)
    nrows = NS * ROWS_PER_TILE
    peidx = jnp.stack([pcol.reshape(NC, nrows, EROW),
                       plr.reshape(NC, nrows, EROW)], axis=2)
    pvals = pval.reshape(NC, nrows, EROW)

    e0 = jnp.concatenate([user_emb, item_emb], axis=0)
    e1 = _sc_layer(e0, peidx, pvals, pcount)
    e2 = _sc_layer(e1, peidx, pvals, pcount)
    e3 = _sc_layer(e2, peidx, pvals, pcount)

    light = _mean4(e0, e1, e2, e3)
    users_id_emb = light[:N_USERS]
    items_id_emb = light[N_USERS:]
    user_lm = _user_lm(phi3, phi4, phi2, user_lm_T)
    return (users_id_emb, items_id_emb, user_lm, item_lm)
